# baseline (speedup 1.0000x reference)
import jax
import jax.numpy as jnp
from jax import lax
from jax.experimental import pallas as pl
from jax.experimental.pallas import tpu as pltpu

_CH = 2048
_D = 4
_S = 4


def _proj_body(hl_ref, W2_ref, b2_ref, proj_ref):
    proj = lax.dot_general(hl_ref[...], W2_ref[...],
                           (((1,), (1,)), ((), ())),
                           precision=lax.Precision.HIGHEST,
                           preferred_element_type=jnp.float32)
    proj_ref[...] = proj + b2_ref[...]


def _chunk_copies(q_hbm, buf_ref, sem, chunk, slot):
    sub = _CH // _S
    return [pltpu.make_async_copy(
        q_hbm.at[pl.ds(chunk * _CH + s * sub, sub), :],
        buf_ref.at[slot, pl.ds(s * sub, sub), :],
        sem.at[slot]) for s in range(_S)]


def _start_chunk(q_hbm, buf_ref, sem, chunk, slot):
    for c in _chunk_copies(q_hbm, buf_ref, sem, chunk, slot):
        c.start()


def _wait_chunk(q_hbm, buf_ref, sem, chunk, slot):
    for c in _chunk_copies(q_hbm, buf_ref, sem, chunk, slot):
        c.wait()


def _stream_body(scal_ref, proj_ref, vW_ref, g_ref, q_hbm,
                 logits_ref, idx_ref, buf_ref, m_ref, mi_ref, sem):
    i = pl.program_id(0)
    nb = pl.num_programs(0)
    slot = lax.rem(i, _D)

    @pl.when(i == 0)
    def _prologue():
        m_ref[0] = -jnp.inf
        mi_ref[0] = 0
        for d in range(_D):
            _start_chunk(q_hbm, buf_ref, sem, d, d)

    temp = scal_ref[0, 0]
    tanh_c = scal_ref[0, 1]
    vb_c = scal_ref[0, 2]

    _wait_chunk(q_hbm, buf_ref, sem, i, slot)

    hidden = jnp.tanh(buf_ref[slot] + proj_ref[...])
    vW16 = vW_ref[...].astype(jnp.bfloat16)
    row = lax.dot_general(vW16, hidden.astype(jnp.bfloat16),
                          (((1,), (1,)), ((), ())),
                          preferred_element_type=jnp.float32)
    logits_blk = tanh_c * jnp.tanh((row + vb_c) / temp)
    logits_ref[0, pl.ds(i * _CH, _CH)] = logits_blk[0]

    score = logits_blk + g_ref[0, pl.ds(i * _CH, _CH)]
    local_max = jnp.max(score)
    ids = lax.broadcasted_iota(jnp.int32, score.shape, 1)
    local_arg = jnp.min(jnp.where(score == local_max, ids, _CH))
    cur_m = m_ref[0]
    upd = local_max > cur_m
    m_ref[0] = jnp.where(upd, local_max, cur_m)
    mi_ref[0] = jnp.where(upd, i * _CH + local_arg, mi_ref[0])

    @pl.when(i + _D < nb)
    def _refill():
        _start_chunk(q_hbm, buf_ref, sem, i + _D, slot)

    @pl.when(i == nb - 1)
    def _fin():
        idx_ref[0, 0] = mi_ref[0]


def kernel(h, query, W2, b2, vW, vb, temperature, tanh_constant):
    N, L = query.shape
    nsteps = N // _CH
    hl = h[-1].reshape(1, L)
    g = jax.random.gumbel(jax.random.key(42), (1, N), jnp.float32)
    scal = jnp.stack([jnp.asarray(temperature, jnp.float32),
                      jnp.asarray(tanh_constant, jnp.float32),
                      vb.astype(jnp.float32)[0],
                      jnp.float32(0)]).reshape(1, 4)

    proj = pl.pallas_call(
        _proj_body,
        out_shape=jax.ShapeDtypeStruct((1, L), jnp.float32),
    )(hl, W2, b2.reshape(1, L))

    logits, idx = pl.pallas_call(
        _stream_body,
        grid=(nsteps,),
        in_specs=[
            pl.BlockSpec(memory_space=pltpu.SMEM),
            pl.BlockSpec((1, L), lambda i: (0, 0)),
            pl.BlockSpec((1, L), lambda i: (0, 0)),
            pl.BlockSpec(memory_space=pltpu.VMEM),
            pl.BlockSpec(memory_space=pl.ANY),
        ],
        out_specs=[
            pl.BlockSpec(memory_space=pltpu.VMEM),
            pl.BlockSpec((1, 1), lambda i: (0, 0),
                         memory_space=pltpu.SMEM),
        ],
        out_shape=[
            jax.ShapeDtypeStruct((1, N), jnp.float32),
            jax.ShapeDtypeStruct((1, 1), jnp.int32),
        ],
        scratch_shapes=[
            pltpu.VMEM((_D, _CH, L), jnp.float32),
            pltpu.SMEM((1,), jnp.float32),
            pltpu.SMEM((1,), jnp.int32),
            pltpu.SemaphoreType.DMA((_D,)),
        ],
    )(scal, proj, vW, g, query)

    return (idx[0, 0], logits)


# speedup vs baseline: 1.0490x; 1.0490x over previous
import jax
import jax.numpy as jnp
from jax import lax
from jax.experimental import pallas as pl
from jax.experimental.pallas import tpu as pltpu

_CH = 2048
_D = 4
_S = 4


def _proj_body(hl_ref, W2_ref, b2_ref, proj_ref):
    proj = lax.dot_general(hl_ref[...], W2_ref[...],
                           (((1,), (1,)), ((), ())),
                           preferred_element_type=jnp.float32)
    proj_ref[...] = proj + b2_ref[...]


def _chunk_copies(q_hbm, buf_ref, sem, chunk, slot):
    sub = _CH // _S
    return [pltpu.make_async_copy(
        q_hbm.at[pl.ds(chunk * _CH + s * sub, sub), :],
        buf_ref.at[slot, pl.ds(s * sub, sub), :],
        sem.at[slot]) for s in range(_S)]


def _start_chunk(q_hbm, buf_ref, sem, chunk, slot):
    for c in _chunk_copies(q_hbm, buf_ref, sem, chunk, slot):
        c.start()


def _wait_chunk(q_hbm, buf_ref, sem, chunk, slot):
    for c in _chunk_copies(q_hbm, buf_ref, sem, chunk, slot):
        c.wait()


def _stream_body(scal_ref, proj_ref, vW_ref, g_ref, q_hbm,
                 logits_ref, idx_ref, buf_ref, m_ref, mi_ref, sem):
    i = pl.program_id(0)
    nb = pl.num_programs(0)
    slot = lax.rem(i, _D)

    @pl.when(i == 0)
    def _prologue():
        m_ref[0] = -jnp.inf
        mi_ref[0] = 0
        for d in range(_D):
            _start_chunk(q_hbm, buf_ref, sem, d, d)

    temp = scal_ref[0, 0]
    tanh_c = scal_ref[0, 1]
    vb_c = scal_ref[0, 2]

    _wait_chunk(q_hbm, buf_ref, sem, i, slot)

    hidden = jnp.tanh(buf_ref[slot] + proj_ref[...])
    vW16 = vW_ref[...].astype(jnp.bfloat16)
    row = lax.dot_general(vW16, hidden.astype(jnp.bfloat16),
                          (((1,), (1,)), ((), ())),
                          preferred_element_type=jnp.float32)
    logits_blk = tanh_c * jnp.tanh((row + vb_c) / temp)
    logits_ref[0, pl.ds(i * _CH, _CH)] = logits_blk[0]

    score = logits_blk + g_ref[0, pl.ds(i * _CH, _CH)]
    local_max = jnp.max(score)
    ids = lax.broadcasted_iota(jnp.int32, score.shape, 1)
    local_arg = jnp.min(jnp.where(score == local_max, ids, _CH))
    cur_m = m_ref[0]
    upd = local_max > cur_m
    m_ref[0] = jnp.where(upd, local_max, cur_m)
    mi_ref[0] = jnp.where(upd, i * _CH + local_arg, mi_ref[0])

    @pl.when(i + _D < nb)
    def _refill():
        _start_chunk(q_hbm, buf_ref, sem, i + _D, slot)

    @pl.when(i == nb - 1)
    def _fin():
        idx_ref[0, 0] = mi_ref[0]


def kernel(h, query, W2, b2, vW, vb, temperature, tanh_constant):
    N, L = query.shape
    nsteps = N // _CH
    hl = h[-1].reshape(1, L)
    g = jax.random.gumbel(jax.random.key(42), (1, N), jnp.float32)
    scal = jnp.stack([jnp.asarray(temperature, jnp.float32),
                      jnp.asarray(tanh_constant, jnp.float32),
                      vb.astype(jnp.float32)[0],
                      jnp.float32(0)]).reshape(1, 4)

    proj = pl.pallas_call(
        _proj_body,
        out_shape=jax.ShapeDtypeStruct((1, L), jnp.float32),
    )(hl, W2, b2.reshape(1, L))

    logits, idx = pl.pallas_call(
        _stream_body,
        grid=(nsteps,),
        in_specs=[
            pl.BlockSpec(memory_space=pltpu.SMEM),
            pl.BlockSpec((1, L), lambda i: (0, 0)),
            pl.BlockSpec((1, L), lambda i: (0, 0)),
            pl.BlockSpec(memory_space=pltpu.VMEM),
            pl.BlockSpec(memory_space=pl.ANY),
        ],
        out_specs=[
            pl.BlockSpec(memory_space=pltpu.VMEM),
            pl.BlockSpec((1, 1), lambda i: (0, 0),
                         memory_space=pltpu.SMEM),
        ],
        out_shape=[
            jax.ShapeDtypeStruct((1, N), jnp.float32),
            jax.ShapeDtypeStruct((1, 1), jnp.int32),
        ],
        scratch_shapes=[
            pltpu.VMEM((_D, _CH, L), jnp.float32),
            pltpu.SMEM((1,), jnp.float32),
            pltpu.SMEM((1,), jnp.int32),
            pltpu.SemaphoreType.DMA((_D,)),
        ],
    )(scal, proj, vW, g, query)

    return (idx[0, 0], logits)
